# baseline (speedup 1.0000x reference)
import functools

import jax
import jax.numpy as jnp
from jax import lax
from jax.experimental import pallas as pl
from jax.experimental.pallas import tpu as pltpu
from jax.experimental.pallas import tpu_sc as plsc

VOCAB = 100000
DIM = 512
SEQ = 200
BATCH = 1024

NC = 2
NS = 16
NW = NC * NS

N = SEQ * BATCH
PER_W = N // NW
CHUNK = 100
NCH = PER_W // CHUNK


def _add_body(a_ref, b_ref, o_ref):
    o_ref[...] = a_ref[...] + b_ref[...]


def _sum_tables(W1, W2):
    BV = 2000
    return pl.pallas_call(
        _add_body,
        grid=(VOCAB // BV,),
        in_specs=[
            pl.BlockSpec((BV, DIM), lambda i: (i, 0)),
            pl.BlockSpec((BV, DIM), lambda i: (i, 0)),
        ],
        out_specs=pl.BlockSpec((BV, DIM), lambda i: (i, 0)),
        out_shape=jax.ShapeDtypeStruct((VOCAB, DIM), jnp.float32),
    )(W1, W2)


def _gather_body(tbl_hbm, idx_hbm, out_hbm, idx_v, buf0, buf1, gsem0, gsem1,
                 wsem0, wsem1):
    c = lax.axis_index("c")
    s = lax.axis_index("s")
    wid = s * NC + c

    pltpu.sync_copy(idx_hbm.at[wid], idx_v)

    bufs = (buf0, buf1)
    gsems = (gsem0, gsem1)
    wsems = (wsem0, wsem1)

    def start_gather(j, b):
        pltpu.async_copy(tbl_hbm.at[idx_v.at[j]], bufs[b], gsems[b])

    def wait_gather(j, b):
        pltpu.make_async_copy(tbl_hbm.at[idx_v.at[j]], bufs[b], gsems[b]).wait()

    def start_write(j, b):
        pltpu.async_copy(bufs[b], out_hbm.at[wid, j], wsems[b])

    def wait_write(j, b):
        pltpu.make_async_copy(bufs[b], out_hbm.at[wid, j], wsems[b]).wait()

    start_gather(0, 0)
    start_gather(1, 1)
    wait_gather(0, 0)
    start_write(0, 0)

    def steady(jj, carry):
        for b, j in ((1, 2 * jj + 1), (0, 2 * jj + 2)):
            wait_write(j - 1, 1 - b)
            start_gather(j + 1, 1 - b)
            wait_gather(j, b)
            start_write(j, b)
        return carry

    lax.fori_loop(0, (NCH - 2) // 2, steady, 0)

    wait_write(NCH - 2, 0)
    wait_gather(NCH - 1, 1)
    start_write(NCH - 1, 1)
    wait_write(NCH - 1, 1)


def _sc_gather(tbl, idx):
    mesh = plsc.VectorSubcoreMesh(
        core_axis_name="c", subcore_axis_name="s", num_cores=NC, num_subcores=NS
    )
    f = pl.kernel(
        _gather_body,
        out_type=jax.ShapeDtypeStruct((NW, NCH, CHUNK, DIM), jnp.float32),
        mesh=mesh,
        scratch_types=[
            pltpu.VMEM((NCH, CHUNK), jnp.int32),
            pltpu.VMEM((CHUNK, DIM), jnp.float32),
            pltpu.VMEM((CHUNK, DIM), jnp.float32),
            pltpu.SemaphoreType.DMA,
            pltpu.SemaphoreType.DMA,
            pltpu.SemaphoreType.DMA,
            pltpu.SemaphoreType.DMA,
        ],
    )
    return f(tbl, idx)


def kernel(x, W1, W2):
    tbl = _sum_tables(W1, W2)
    idx = x.astype(jnp.int32).reshape(NW, NCH, CHUNK)
    out = _sc_gather(tbl, idx)
    return out.reshape(SEQ, BATCH, DIM)


# speedup vs baseline: 1.7605x; 1.7605x over previous
import functools

import jax
import jax.numpy as jnp
from jax import lax
from jax.experimental import pallas as pl
from jax.experimental.pallas import tpu as pltpu
from jax.experimental.pallas import tpu_sc as plsc

VOCAB = 100000
DIM = 512
SEQ = 200
BATCH = 1024

NC = 2
NS = 16
NW = NC * NS

N = SEQ * BATCH
PER_W = N // NW
CHUNK = 80
NCH = PER_W // CHUNK


def _add_body(a_ref, b_ref, o_ref):
    o_ref[...] = a_ref[...] + b_ref[...]


def _sum_tables(W1, W2):
    BV = 2000
    return pl.pallas_call(
        _add_body,
        grid=(VOCAB // BV,),
        in_specs=[
            pl.BlockSpec((BV, DIM), lambda i: (i, 0)),
            pl.BlockSpec((BV, DIM), lambda i: (i, 0)),
        ],
        out_specs=pl.BlockSpec((BV, DIM), lambda i: (i, 0)),
        out_shape=jax.ShapeDtypeStruct((VOCAB, DIM), jnp.float32),
    )(W1, W2)


def _gather_body(tbl_hbm, idx_hbm, out_hbm, idx_v, buf0, buf1, gsem0, gsem1,
                 wsem0, wsem1):
    c = lax.axis_index("c")
    s = lax.axis_index("s")
    wid = s * NC + c

    pltpu.sync_copy(idx_hbm.at[wid], idx_v)

    bufs = (buf0, buf1)
    gsems = (gsem0, gsem1)
    wsems = (wsem0, wsem1)

    def start_gather(j, b):
        pltpu.async_copy(tbl_hbm.at[idx_v.at[j]], bufs[b], gsems[b])

    def wait_gather(j, b):
        pltpu.make_async_copy(tbl_hbm.at[idx_v.at[j]], bufs[b], gsems[b]).wait()

    def start_write(j, b):
        pltpu.async_copy(bufs[b], out_hbm.at[wid, j], wsems[b])

    def wait_write(j, b):
        pltpu.make_async_copy(bufs[b], out_hbm.at[wid, j], wsems[b]).wait()

    start_gather(0, 0)
    start_gather(1, 1)
    wait_gather(0, 0)
    start_write(0, 0)

    def steady(jj, carry):
        for b, j in ((1, 2 * jj + 1), (0, 2 * jj + 2)):
            wait_write(j - 1, 1 - b)
            start_gather(j + 1, 1 - b)
            wait_gather(j, b)
            start_write(j, b)
        return carry

    lax.fori_loop(0, (NCH - 2) // 2, steady, 0)

    wait_write(NCH - 2, 0)
    wait_gather(NCH - 1, 1)
    start_write(NCH - 1, 1)
    wait_write(NCH - 1, 1)


def _sc_gather(tbl, idx):
    mesh = plsc.VectorSubcoreMesh(
        core_axis_name="c", subcore_axis_name="s", num_cores=NC, num_subcores=NS
    )
    f = pl.kernel(
        _gather_body,
        out_type=jax.ShapeDtypeStruct((NW, NCH, CHUNK, DIM), jnp.float32),
        mesh=mesh,
        scratch_types=[
            pltpu.VMEM((NCH, CHUNK), jnp.int32),
            pltpu.VMEM((CHUNK, DIM), jnp.float32),
            pltpu.VMEM((CHUNK, DIM), jnp.float32),
            pltpu.SemaphoreType.DMA,
            pltpu.SemaphoreType.DMA,
            pltpu.SemaphoreType.DMA,
            pltpu.SemaphoreType.DMA,
        ],
    )
    return f(tbl, idx)


def kernel(x, W1, W2):
    tbl = _sum_tables(W1, W2)
    idx = x.astype(jnp.int32).reshape(NW, NCH, CHUNK)
    out = _sc_gather(tbl, idx)
    return out.reshape(SEQ, BATCH, DIM)
